# baseline (speedup 1.0000x reference)
import functools

import jax
import jax.numpy as jnp
from jax import lax
from jax.experimental import pallas as pl
from jax.experimental.pallas import tpu as pltpu
from jax.experimental.pallas import tpu_sc as plsc

B, C, H, W = 4, 384, 224, 224
K = 8192
HID = 256
CB = 32
NCB = C // CB
GRID = B * NCB
BC = B * C
NW = 32
PPW = BC // NW


def _sum_body(x_ref, o_ref):
    o_ref[0, 0, 0, :] = jnp.sum(x_ref[...], axis=(0, 2, 3))


def _scales_body(sums_ref, protos_ref, w1_ref, b1_ref, w2_ref, b2_ref, o_ref):
    s = sums_ref[...] * (1.0 / (H * W))
    protos = protos_ref[...]
    cross = jax.lax.dot_general(
        s, protos, (((1,), (1,)), ((), ())),
        preferred_element_type=jnp.float32)
    psq = jnp.sum(protos * protos, axis=1)
    d2 = psq[None, :] - 2.0 * cross
    idx = jnp.argmin(d2, axis=1)
    onehot = (jax.lax.broadcasted_iota(jnp.int32, (B, K), 1)
              == idx[:, None]).astype(jnp.float32)
    matched = jax.lax.dot_general(
        onehot, protos, (((1,), (0,)), ((), ())),
        preferred_element_type=jnp.float32)
    h = jax.lax.dot_general(
        matched, w1_ref[...], (((1,), (1,)), ((), ())),
        preferred_element_type=jnp.float32) + b1_ref[...]
    h = jnp.maximum(h, 0.0)
    g = jax.lax.dot_general(
        h, w2_ref[...], (((1,), (1,)), ((), ())),
        preferred_element_type=jnp.float32) + b2_ref[...]
    o_ref[...] = jax.nn.sigmoid(g)


def _sc_mul_body(x_hbm, s_hbm, y_hbm, s_v, b0, b1, sin0, sin1, sout0, sout1):
    wid = lax.axis_index("s") * 2 + lax.axis_index("c")
    base = wid * PPW
    pltpu.sync_copy(s_hbm.at[pl.ds(base * 16, PPW * 16)], s_v)

    bufs = (b0, b1)
    sins = (sin0, sin1)
    souts = (sout0, sout1)

    in_copies = [None, None]
    out_copies = [None, None]

    in_copies[0] = pltpu.async_copy(x_hbm.at[base + 0], b0, sin0)
    for p in range(PPW):
        par = p % 2
        buf = bufs[par]
        in_copies[par].wait()
        svec = s_v[pl.ds(p * 16, 16)]

        def body(r, _):
            for j in range(W // 16):
                sl = pl.ds(j * 16, 16)
                buf[r, sl] = buf[r, sl] * svec
            return 0

        lax.fori_loop(0, H, body, 0, unroll=2)

        if p + 1 < PPW:
            nxt = (p + 1) % 2
            if out_copies[nxt] is not None:
                out_copies[nxt].wait()
            in_copies[nxt] = pltpu.async_copy(
                x_hbm.at[base + p + 1], bufs[nxt], sins[nxt])
        out_copies[par] = pltpu.async_copy(buf, y_hbm.at[base + p], souts[par])

    out_copies[(PPW - 1) % 2].wait()
    out_copies[(PPW - 2) % 2].wait()


@jax.jit
def kernel(x, prototypes, W1, b1, W2, b2):
    sums = pl.pallas_call(
        _sum_body,
        grid=(GRID,),
        in_specs=[pl.BlockSpec((1, CB, H, W), lambda i: (i // NCB, i % NCB, 0, 0))],
        out_specs=pl.BlockSpec((1, 1, 1, CB), lambda i: (i // NCB, i % NCB, 0, 0)),
        out_shape=jax.ShapeDtypeStruct((B, NCB, 1, CB), jnp.float32),
        compiler_params=pltpu.CompilerParams(
            dimension_semantics=("parallel",)),
    )(x)

    scales = pl.pallas_call(
        _scales_body,
        out_shape=jax.ShapeDtypeStruct((B, C), jnp.float32),
    )(sums.reshape(B, C), prototypes, W1, b1[None, :], W2, b2[None, :])

    x3 = x.reshape(BC, H, W)
    sc_mul = functools.partial(
        pl.kernel,
        mesh=plsc.VectorSubcoreMesh(core_axis_name="c", subcore_axis_name="s"),
        compiler_params=pltpu.CompilerParams(needs_layout_passes=False),
        out_type=jax.ShapeDtypeStruct((BC, H, W), jnp.float32),
        scratch_types=[
            pltpu.VMEM((PPW * 16,), jnp.float32),
            pltpu.VMEM((H, W), jnp.float32),
            pltpu.VMEM((H, W), jnp.float32),
            pltpu.SemaphoreType.DMA,
            pltpu.SemaphoreType.DMA,
            pltpu.SemaphoreType.DMA,
            pltpu.SemaphoreType.DMA,
        ],
    )(_sc_mul_body)
    s_exp = jnp.broadcast_to(scales.reshape(BC, 1), (BC, 16)).reshape(BC * 16)
    y3 = sc_mul(x3, s_exp)
    return y3.reshape(B, C, H, W)
